# baseline (speedup 1.0000x reference)
import jax
import jax.numpy as jnp
from jax import lax
from jax.experimental import pallas as pl
from jax.experimental.pallas import tpu as pltpu
from jax.experimental.pallas import tpu_sc as plsc

BATCH = 16384
NF = 8
NM = 32
NROWS = 1_000_000

_info = plsc.get_sparse_core_info()
_NC, _NS = _info.num_cores, _info.num_subcores
NW = _NC * _NS
B_PER_W = BATCH // NW
CHUNK = 128
NCHUNK = B_PER_W // CHUNK


def _xpose_body(in_ref, out_ref):
    f = in_ref.shape[0]
    eye = jnp.eye(f, dtype=jnp.float32)
    out_ref[...] = lax.dot_general(in_ref[...], eye, (((0,), (0,)), ((), ())),
                                   preferred_element_type=jnp.float32)


def _xpose(table_t, bn):
    f, n = table_t.shape
    return pl.pallas_call(
        _xpose_body,
        grid=(pl.cdiv(n, bn),),
        in_specs=[pl.BlockSpec((f, bn), lambda i: (0, i))],
        out_specs=pl.BlockSpec((bn, f), lambda i: (i, 0)),
        out_shape=jax.ShapeDtypeStruct((n, f), jnp.float32),
        name="neumf_xpose",
    )(table_t)


def _sc_gather_body(users_hbm, items_hbm, ug_t, ig_t, um_t, im_t,
                    ug_o, ig_o, eu_o, ei_o,
                    idx_u, idx_i, ug_v, ig_v, eu_v, ei_v, sem):
    wid = lax.axis_index("s") * _NC + lax.axis_index("c")
    base = wid * B_PER_W
    pltpu.sync_copy(users_hbm.at[wid], idx_u)
    pltpu.sync_copy(items_hbm.at[wid], idx_i)
    copies = []
    for j in range(NCHUNK):
        dst = pl.ds(j * CHUNK, CHUNK)
        copies.append(pltpu.async_copy(ug_t.at[idx_u.at[j]], ug_v.at[dst], sem))
        copies.append(pltpu.async_copy(ig_t.at[idx_i.at[j]], ig_v.at[dst], sem))
        copies.append(pltpu.async_copy(um_t.at[idx_u.at[j]], eu_v.at[dst], sem))
        copies.append(pltpu.async_copy(im_t.at[idx_i.at[j]], ei_v.at[dst], sem))
    for c in copies:
        c.wait()
    out_rows = pl.ds(base, B_PER_W)
    pltpu.sync_copy(ug_v, ug_o.at[out_rows])
    pltpu.sync_copy(ig_v, ig_o.at[out_rows])
    pltpu.sync_copy(eu_v, eu_o.at[out_rows])
    pltpu.sync_copy(ei_v, ei_o.at[out_rows])


_sc_gather = pl.kernel(
    _sc_gather_body,
    out_type=(
        jax.ShapeDtypeStruct((BATCH, NF), jnp.float32),
        jax.ShapeDtypeStruct((BATCH, NF), jnp.float32),
        jax.ShapeDtypeStruct((BATCH, NM), jnp.float32),
        jax.ShapeDtypeStruct((BATCH, NM), jnp.float32),
    ),
    mesh=plsc.VectorSubcoreMesh(core_axis_name="c", subcore_axis_name="s"),
    scratch_types=[
        pltpu.VMEM((NCHUNK, CHUNK), jnp.int32),
        pltpu.VMEM((NCHUNK, CHUNK), jnp.int32),
        pltpu.VMEM((B_PER_W, NF), jnp.float32),
        pltpu.VMEM((B_PER_W, NF), jnp.float32),
        pltpu.VMEM((B_PER_W, NM), jnp.float32),
        pltpu.VMEM((B_PER_W, NM), jnp.float32),
        pltpu.SemaphoreType.DMA,
    ],
    compiler_params=pltpu.CompilerParams(use_tc_tiling_on_sc=False),
    name="neumf_sc_gather",
)


def _tc_mlp_body(ug_ref, ig_ref, eu_ref, ei_ref, w1_ref, b1_ref, w2_ref,
                 b2_ref, w3_ref, b3_ref, wl_ref, bl_ref, out_ref):
    dn = (((1,), (1,)), ((), ()))
    f32 = jnp.float32
    w1 = w1_ref[...]
    h1 = lax.dot_general(eu_ref[...], w1[:, :NM], dn, preferred_element_type=f32)
    h1 = h1 + lax.dot_general(ei_ref[...], w1[:, NM:], dn, preferred_element_type=f32)
    h1 = jnp.maximum(h1 + b1_ref[...], 0.0)
    h2 = lax.dot_general(h1, w2_ref[...], dn, preferred_element_type=f32)
    h2 = jnp.maximum(h2 + b2_ref[...], 0.0)
    h3 = lax.dot_general(h2, w3_ref[...], dn, preferred_element_type=f32)
    h3 = jnp.maximum(h3 + b3_ref[...], 0.0)
    gmf = ug_ref[...] * ig_ref[...]
    wl = wl_ref[...]
    out = lax.dot_general(gmf, wl[:, :NF], dn, preferred_element_type=f32)
    out = out + lax.dot_general(h3, wl[:, NF:], dn, preferred_element_type=f32)
    out_ref[...] = out + bl_ref[...]


def _tc_mlp(ug, ig, eu, ei, W1, b1, W2, b2, W3, b3, Wl, bl):
    bs = 2048
    grid = (BATCH // bs,)
    row = lambda i: (i, 0)
    rep = lambda i: (0, 0)
    return pl.pallas_call(
        _tc_mlp_body,
        grid=grid,
        in_specs=[
            pl.BlockSpec((bs, NF), row),
            pl.BlockSpec((bs, NF), row),
            pl.BlockSpec((bs, NM), row),
            pl.BlockSpec((bs, NM), row),
            pl.BlockSpec(W1.shape, rep),
            pl.BlockSpec((1, NM), rep),
            pl.BlockSpec(W2.shape, rep),
            pl.BlockSpec((1, 16), rep),
            pl.BlockSpec(W3.shape, rep),
            pl.BlockSpec((1, NF), rep),
            pl.BlockSpec((1, 16), rep),
            pl.BlockSpec((1, 1), rep),
        ],
        out_specs=pl.BlockSpec((bs, 1), row),
        out_shape=jax.ShapeDtypeStruct((BATCH, 1), jnp.float32),
        name="neumf_tc_mlp",
    )(ug, ig, eu, ei, W1, b1, W2, b2, W3, b3, Wl, bl)


def kernel(users, items, Ug, Ig, Um, Im, W1, b1, W2, b2, W3, b3, Wl, bl):
    u3 = users.astype(jnp.int32).reshape(NW, NCHUNK, CHUNK)
    i3 = items.astype(jnp.int32).reshape(NW, NCHUNK, CHUNK)
    bn = 16384
    ug_rm = _xpose(Ug.T, bn)
    ig_rm = _xpose(Ig.T, bn)
    um_rm = _xpose(Um.T, bn)
    im_rm = _xpose(Im.T, bn)
    ug, ig, eu, ei = _sc_gather(u3, i3, ug_rm, ig_rm, um_rm, im_rm)
    out = _tc_mlp(ug, ig, eu, ei,
                  W1, b1.reshape(1, -1), W2, b2.reshape(1, -1),
                  W3, b3.reshape(1, -1), Wl, bl.reshape(1, 1))
    return out.reshape(-1)


# speedup vs baseline: 1.3887x; 1.3887x over previous
import jax
import jax.numpy as jnp
from jax import lax
from jax.experimental import pallas as pl
from jax.experimental.pallas import tpu as pltpu
from jax.experimental.pallas import tpu_sc as plsc

BATCH = 16384
NF = 8
NM = 32
NS = NF + NM
NROWS = 1_000_000

_info = plsc.get_sparse_core_info()
_NC, _NSUB = _info.num_cores, _info.num_subcores
NW = _NC * _NSUB
B_PER_W = BATCH // NW
CHUNK = 128
NCHUNK = B_PER_W // CHUNK


def _pack_body(ug_ref, um_ref, ig_ref, im_ref, out_ref):
    def t(ref):
        f = ref.shape[0]
        eye = jnp.eye(f, dtype=jnp.float32)
        return lax.dot_general(ref[...], eye, (((0,), (0,)), ((), ())),
                               preferred_element_type=jnp.float32)
    out_ref[:, 0:NF] = t(ug_ref)
    out_ref[:, NF:NS] = t(um_ref)
    out_ref[:, NS:NS + NF] = t(ig_ref)
    out_ref[:, NS + NF:] = t(im_ref)


def _pack(ug_t, um_t, ig_t, im_t):
    bn = 16384
    col = lambda i: (0, i)
    return pl.pallas_call(
        _pack_body,
        grid=(pl.cdiv(NROWS, bn),),
        in_specs=[
            pl.BlockSpec((NF, bn), col),
            pl.BlockSpec((NM, bn), col),
            pl.BlockSpec((NF, bn), col),
            pl.BlockSpec((NM, bn), col),
        ],
        out_specs=pl.BlockSpec((bn, 2 * NS), lambda i: (i, 0)),
        out_shape=jax.ShapeDtypeStruct((NROWS, 2 * NS), jnp.float32),
        name="neumf_pack",
    )(ug_t, um_t, ig_t, im_t)


def _sc_gather_body(users_hbm, items_hbm, tab,
                    u_o, i_o,
                    idx_u, idx_i, u_v, i_v, sem):
    wid = lax.axis_index("s") * _NC + lax.axis_index("c")
    base = wid * B_PER_W
    pltpu.sync_copy(users_hbm.at[wid], idx_u)
    pltpu.sync_copy(items_hbm.at[wid], idx_i)
    copies = []
    for j in range(NCHUNK):
        dst = pl.ds(j * CHUNK, CHUNK)
        copies.append(pltpu.async_copy(tab.at[idx_u.at[j]], u_v.at[dst], sem))
        copies.append(pltpu.async_copy(tab.at[idx_i.at[j]], i_v.at[dst], sem))
    for c in copies:
        c.wait()
    out_rows = pl.ds(base, B_PER_W)
    pltpu.sync_copy(u_v, u_o.at[out_rows])
    pltpu.sync_copy(i_v, i_o.at[out_rows])


_sc_gather = pl.kernel(
    _sc_gather_body,
    out_type=(
        jax.ShapeDtypeStruct((BATCH, 2 * NS), jnp.float32),
        jax.ShapeDtypeStruct((BATCH, 2 * NS), jnp.float32),
    ),
    mesh=plsc.VectorSubcoreMesh(core_axis_name="c", subcore_axis_name="s"),
    scratch_types=[
        pltpu.VMEM((NCHUNK, CHUNK), jnp.int32),
        pltpu.VMEM((NCHUNK, CHUNK), jnp.int32),
        pltpu.VMEM((B_PER_W, 2 * NS), jnp.float32),
        pltpu.VMEM((B_PER_W, 2 * NS), jnp.float32),
        pltpu.SemaphoreType.DMA,
    ],
    compiler_params=pltpu.CompilerParams(use_tc_tiling_on_sc=False),
    name="neumf_sc_gather",
)


def _tc_mlp_body(u_ref, i_ref, w1_ref, b1_ref, w2_ref,
                 b2_ref, w3_ref, b3_ref, wl_ref, bl_ref, out_ref):
    dn = (((1,), (1,)), ((), ()))
    f32 = jnp.float32
    u = u_ref[...]
    i = i_ref[...]
    w1 = w1_ref[...]
    h1 = lax.dot_general(u[:, NF:NS], w1[:, :NM], dn, preferred_element_type=f32)
    h1 = h1 + lax.dot_general(i[:, NS + NF:], w1[:, NM:], dn, preferred_element_type=f32)
    h1 = jnp.maximum(h1 + b1_ref[...], 0.0)
    h2 = lax.dot_general(h1, w2_ref[...], dn, preferred_element_type=f32)
    h2 = jnp.maximum(h2 + b2_ref[...], 0.0)
    h3 = lax.dot_general(h2, w3_ref[...], dn, preferred_element_type=f32)
    h3 = jnp.maximum(h3 + b3_ref[...], 0.0)
    gmf = u[:, :NF] * i[:, NS:NS + NF]
    wl = wl_ref[...]
    out = lax.dot_general(gmf, wl[:, :NF], dn, preferred_element_type=f32)
    out = out + lax.dot_general(h3, wl[:, NF:], dn, preferred_element_type=f32)
    out_ref[...] = out + bl_ref[...]


def _tc_mlp(u, i, W1, b1, W2, b2, W3, b3, Wl, bl):
    bs = 2048
    grid = (BATCH // bs,)
    row = lambda i: (i, 0)
    rep = lambda i: (0, 0)
    return pl.pallas_call(
        _tc_mlp_body,
        grid=grid,
        in_specs=[
            pl.BlockSpec((bs, 2 * NS), row),
            pl.BlockSpec((bs, 2 * NS), row),
            pl.BlockSpec(W1.shape, rep),
            pl.BlockSpec((1, NM), rep),
            pl.BlockSpec(W2.shape, rep),
            pl.BlockSpec((1, 16), rep),
            pl.BlockSpec(W3.shape, rep),
            pl.BlockSpec((1, NF), rep),
            pl.BlockSpec((1, 16), rep),
            pl.BlockSpec((1, 1), rep),
        ],
        out_specs=pl.BlockSpec((bs, 1), row),
        out_shape=jax.ShapeDtypeStruct((BATCH, 1), jnp.float32),
        name="neumf_tc_mlp",
    )(u, i, W1, b1, W2, b2, W3, b3, Wl, bl)


def kernel(users, items, Ug, Ig, Um, Im, W1, b1, W2, b2, W3, b3, Wl, bl):
    u3 = users.astype(jnp.int32).reshape(NW, NCHUNK, CHUNK)
    i3 = items.astype(jnp.int32).reshape(NW, NCHUNK, CHUNK)
    tab = _pack(Ug.T, Um.T, Ig.T, Im.T)
    u_rows, i_rows = _sc_gather(u3, i3, tab)
    out = _tc_mlp(u_rows, i_rows,
                  W1, b1.reshape(1, -1), W2, b2.reshape(1, -1),
                  W3, b3.reshape(1, -1), Wl, bl.reshape(1, 1))
    return out.reshape(-1)


# speedup vs baseline: 2.4578x; 1.7698x over previous
import jax
import jax.numpy as jnp
from jax import lax
from jax.experimental import pallas as pl
from jax.experimental.pallas import tpu as pltpu
from jax.experimental.pallas import tpu_sc as plsc

BATCH = 16384
NF = 8
NM = 32
NS = NF + NM
NROWS = 1_000_000

_info = plsc.get_sparse_core_info()
_NC, _NSUB = _info.num_cores, _info.num_subcores
NW = _NC * _NSUB
B_PER_W = BATCH // NW
CHUNK = 128
NCHUNK = B_PER_W // CHUNK


def _pack_body(ug_ref, um_ref, ig_ref, im_ref, out_ref):
    eye = jnp.eye(2 * NS, dtype=jnp.float32)
    cat = jnp.concatenate(
        [ug_ref[...], um_ref[...], ig_ref[...], im_ref[...]], axis=0)
    out_ref[...] = lax.dot_general(cat, eye, (((0,), (0,)), ((), ())),
                                   preferred_element_type=jnp.float32)


def _pack(ug_t, um_t, ig_t, im_t):
    bn = 16384
    col = lambda i: (0, i)
    return pl.pallas_call(
        _pack_body,
        grid=(pl.cdiv(NROWS, bn),),
        in_specs=[
            pl.BlockSpec((NF, bn), col),
            pl.BlockSpec((NM, bn), col),
            pl.BlockSpec((NF, bn), col),
            pl.BlockSpec((NM, bn), col),
        ],
        out_specs=pl.BlockSpec((bn, 2 * NS), lambda i: (i, 0)),
        out_shape=jax.ShapeDtypeStruct((NROWS, 2 * NS), jnp.float32),
        compiler_params=pltpu.CompilerParams(fuse_transposed_lhs_in_matmul=True),
        name="neumf_pack",
    )(ug_t, um_t, ig_t, im_t)


def _sc_gather_body(users_hbm, items_hbm, tab,
                    u_o, i_o,
                    idx_u, idx_i, u_v, i_v, sem):
    wid = lax.axis_index("s") * _NC + lax.axis_index("c")
    base = wid * B_PER_W
    pltpu.sync_copy(users_hbm.at[wid], idx_u)
    pltpu.sync_copy(items_hbm.at[wid], idx_i)
    copies = []
    for j in range(NCHUNK):
        dst = pl.ds(j * CHUNK, CHUNK)
        copies.append(pltpu.async_copy(tab.at[idx_u.at[j]], u_v.at[dst], sem))
        copies.append(pltpu.async_copy(tab.at[idx_i.at[j]], i_v.at[dst], sem))
    for c in copies:
        c.wait()
    out_rows = pl.ds(base, B_PER_W)
    pltpu.sync_copy(u_v, u_o.at[out_rows])
    pltpu.sync_copy(i_v, i_o.at[out_rows])


_sc_gather = pl.kernel(
    _sc_gather_body,
    out_type=(
        jax.ShapeDtypeStruct((BATCH, 2 * NS), jnp.float32),
        jax.ShapeDtypeStruct((BATCH, 2 * NS), jnp.float32),
    ),
    mesh=plsc.VectorSubcoreMesh(core_axis_name="c", subcore_axis_name="s"),
    scratch_types=[
        pltpu.VMEM((NCHUNK, CHUNK), jnp.int32),
        pltpu.VMEM((NCHUNK, CHUNK), jnp.int32),
        pltpu.VMEM((B_PER_W, 2 * NS), jnp.float32),
        pltpu.VMEM((B_PER_W, 2 * NS), jnp.float32),
        pltpu.SemaphoreType.DMA,
    ],
    compiler_params=pltpu.CompilerParams(use_tc_tiling_on_sc=False),
    name="neumf_sc_gather",
)


def _tc_mlp_body(u_ref, i_ref, w1_ref, b1_ref, w2_ref,
                 b2_ref, w3_ref, b3_ref, wl_ref, bl_ref, out_ref):
    dn = (((1,), (1,)), ((), ()))
    f32 = jnp.float32
    u = u_ref[...]
    i = i_ref[...]
    w1 = w1_ref[...]
    h1 = lax.dot_general(u[:, NF:NS], w1[:, :NM], dn, preferred_element_type=f32)
    h1 = h1 + lax.dot_general(i[:, NS + NF:], w1[:, NM:], dn, preferred_element_type=f32)
    h1 = jnp.maximum(h1 + b1_ref[...], 0.0)
    h2 = lax.dot_general(h1, w2_ref[...], dn, preferred_element_type=f32)
    h2 = jnp.maximum(h2 + b2_ref[...], 0.0)
    h3 = lax.dot_general(h2, w3_ref[...], dn, preferred_element_type=f32)
    h3 = jnp.maximum(h3 + b3_ref[...], 0.0)
    gmf = u[:, :NF] * i[:, NS:NS + NF]
    wl = wl_ref[...]
    out = lax.dot_general(gmf, wl[:, :NF], dn, preferred_element_type=f32)
    out = out + lax.dot_general(h3, wl[:, NF:], dn, preferred_element_type=f32)
    out_ref[...] = out + bl_ref[...]


def _tc_mlp(u, i, W1, b1, W2, b2, W3, b3, Wl, bl):
    bs = 2048
    grid = (BATCH // bs,)
    row = lambda i: (i, 0)
    rep = lambda i: (0, 0)
    return pl.pallas_call(
        _tc_mlp_body,
        grid=grid,
        in_specs=[
            pl.BlockSpec((bs, 2 * NS), row),
            pl.BlockSpec((bs, 2 * NS), row),
            pl.BlockSpec(W1.shape, rep),
            pl.BlockSpec((1, NM), rep),
            pl.BlockSpec(W2.shape, rep),
            pl.BlockSpec((1, 16), rep),
            pl.BlockSpec(W3.shape, rep),
            pl.BlockSpec((1, NF), rep),
            pl.BlockSpec((1, 16), rep),
            pl.BlockSpec((1, 1), rep),
        ],
        out_specs=pl.BlockSpec((bs, 1), row),
        out_shape=jax.ShapeDtypeStruct((BATCH, 1), jnp.float32),
        name="neumf_tc_mlp",
    )(u, i, W1, b1, W2, b2, W3, b3, Wl, bl)


def kernel(users, items, Ug, Ig, Um, Im, W1, b1, W2, b2, W3, b3, Wl, bl):
    u3 = users.astype(jnp.int32).reshape(NW, NCHUNK, CHUNK)
    i3 = items.astype(jnp.int32).reshape(NW, NCHUNK, CHUNK)
    tab = _pack(Ug.T, Um.T, Ig.T, Im.T)
    u_rows, i_rows = _sc_gather(u3, i3, tab)
    out = _tc_mlp(u_rows, i_rows,
                  W1, b1.reshape(1, -1), W2, b2.reshape(1, -1),
                  W3, b3.reshape(1, -1), Wl, bl.reshape(1, 1))
    return out.reshape(-1)
